# Initial kernel scaffold; baseline (speedup 1.0000x reference)
import jax
import jax.numpy as jnp
from jax.experimental import pallas as pl


def kernel(x, edge_attr, W, b, ln_w, ln_b, edge_index):
    raise NotImplementedError("write your pallas kernel here")


# speedup vs baseline: 9.7422x; 9.7422x over previous
import dataclasses
import functools

import jax
import jax.numpy as jnp
from jax import lax
from jax.experimental import pallas as pl
from jax.experimental.pallas import tpu as pltpu
from jax.experimental.pallas import tpu_sc as plsc

N = 10000
N_PAD = 10240
E = 320000
D = 128
CHUNK = 128
N_CHUNKS = E // CHUNK
NC = 2
NS = 16
NW = NC * NS
RPT = N_PAD // NS

_sc_mesh = plsc.VectorSubcoreMesh(
    core_axis_name="c", subcore_axis_name="s", num_cores=NC, num_subcores=NS
)


def _lane_bcast(v16, j):
    idx = jnp.full((16, 1), j, dtype=jnp.int32)
    dn = lax.GatherDimensionNumbers(
        offset_dims=(), collapsed_slice_dims=(0,), start_index_map=(0,))
    return lax.gather(v16, idx, dn, (1,),
                      mode=lax.GatherScatterMode.PROMISE_IN_BOUNDS)


def _sc_body(row_hbm, col_hbm, ea_hbm, xw_hbm, part_hbm, dis_hbm,
             col_v, row_v, ea_v, rows_v, deg_v, dis_v, tmp_v, sum_v,
             acc_sh, stage_sh, sem):
    c = lax.axis_index("c")
    s = lax.axis_index("s")
    wid = s * NC + c
    iota16 = lax.iota(jnp.int32, 16)
    zero16 = jnp.zeros((16,), jnp.float32)

    @pl.loop(0, N_PAD // 16)
    def _(i):
        deg_v[pl.ds(i * 16, 16)] = zero16

    @pl.loop(0, CHUNK)
    def _(i):
        @pl.loop(0, D // 16)
        def _(r):
            rows_v[i, pl.ds(r * 16, 16)] = zero16

    @pl.loop(0, RPT // CHUNK)
    def _(i):
        pltpu.sync_copy(rows_v, acc_sh.at[pl.ds(s * RPT + i * CHUNK, CHUNK)])

    @pl.loop(0, (N_CHUNKS + NS - 1) // NS)
    def _(i):
        k = i * NS + s

        @pl.when(k < N_CHUNKS)
        def _():
            pltpu.sync_copy(col_hbm.at[k], col_v)
            pltpu.sync_copy(ea_hbm.at[k], ea_v)

            @pl.loop(0, CHUNK // 16)
            def _(g):
                col16 = col_v[pl.ds(g * 16, 16)]
                ew16 = jnp.abs(plsc.load_gather(ea_v, [iota16 * 4 + g * 64]))
                for j in range(16):
                    plsc.addupdate_scatter(deg_v, [col16], ew16,
                                           mask=iota16 == j)

    pltpu.sync_copy(deg_v, stage_sh.at[s])
    plsc.subcore_barrier()

    @pl.loop(0, RPT // 16)
    def _(r):
        sum_v[pl.ds(r * 16, 16)] = zero16

    @pl.loop(0, NS)
    def _(i):
        pltpu.sync_copy(stage_sh.at[i, pl.ds(s * RPT, RPT)], tmp_v)

        @pl.loop(0, RPT // 16)
        def _(r):
            sum_v[pl.ds(r * 16, 16)] = (sum_v[pl.ds(r * 16, 16)]
                                        + tmp_v[pl.ds(r * 16, 16)])

    @pl.loop(0, RPT // 16)
    def _(r):
        x = sum_v[pl.ds(r * 16, 16)] + 1.0
        ib = plsc.bitcast(x, jnp.int32)
        ib = 0x5F3759DF - lax.shift_right_logical(ib, 1)
        y = plsc.bitcast(ib, jnp.float32)
        xh = 0.5 * x
        y = y * (1.5 - xh * y * y)
        y = y * (1.5 - xh * y * y)
        y = y * (1.5 - xh * y * y)
        y = y * (1.5 - xh * y * y)
        sum_v[pl.ds(r * 16, 16)] = y

    plsc.subcore_barrier()
    pltpu.sync_copy(sum_v, stage_sh.at[0, pl.ds(s * RPT, RPT)])
    plsc.subcore_barrier()
    pltpu.sync_copy(stage_sh.at[0], dis_v)

    @pl.loop(0, (N_CHUNKS + NW - 1) // NW)
    def _(i):
        k = i * NW + wid

        @pl.when(k < N_CHUNKS)
        def _():
            pltpu.sync_copy(row_hbm.at[k], row_v)
            pltpu.sync_copy(col_hbm.at[k], col_v)
            pltpu.sync_copy(ea_hbm.at[k], ea_v)
            pltpu.async_copy(xw_hbm.at[row_v], rows_v, sem).wait()

            @pl.loop(0, CHUNK // 16)
            def _(g):
                row16 = row_v[pl.ds(g * 16, 16)]
                ew16 = jnp.abs(plsc.load_gather(ea_v, [iota16 * 4 + g * 64]))
                disr16 = plsc.load_gather(dis_v, [row16])
                w16 = ew16 * disr16
                for j in range(16):
                    wb = _lane_bcast(w16, j)
                    e = g * 16 + j
                    for r in range(D // 16):
                        rows_v[e, pl.ds(r * 16, 16)] = (
                            rows_v[e, pl.ds(r * 16, 16)] * wb)

            pltpu.sync_copy(rows_v, acc_sh.at[col_v], add=True)

    plsc.subcore_barrier()

    pltpu.sync_copy(acc_sh.at[pl.ds(s * RPT, RPT)],
                    part_hbm.at[c, pl.ds(s * RPT, RPT)])

    @pl.when(c == 0)
    def _():
        pltpu.sync_copy(dis_v.at[pl.ds(s * RPT, RPT)],
                        dis_hbm.at[pl.ds(s * RPT, RPT)])


_sc_cp = pltpu.CompilerParams()
if "needs_layout_passes" in pltpu.CompilerParams.__dataclass_fields__:
    _sc_cp = dataclasses.replace(_sc_cp, needs_layout_passes=False)

_sc_call = functools.partial(
    pl.kernel,
    compiler_params=_sc_cp,
    out_type=[
        jax.ShapeDtypeStruct((NC, N_PAD, D), jnp.float32),
        jax.ShapeDtypeStruct((N_PAD,), jnp.float32),
    ],
    mesh=_sc_mesh,
    scratch_types=[
        pltpu.VMEM((CHUNK,), jnp.int32),
        pltpu.VMEM((CHUNK,), jnp.int32),
        pltpu.VMEM((CHUNK * 4,), jnp.float32),
        pltpu.VMEM((CHUNK, D), jnp.float32),
        pltpu.VMEM((N_PAD,), jnp.float32),
        pltpu.VMEM((N_PAD,), jnp.float32),
        pltpu.VMEM((RPT,), jnp.float32),
        pltpu.VMEM((RPT,), jnp.float32),
        pltpu.VMEM_SHARED((N_PAD, D), jnp.float32),
        pltpu.VMEM_SHARED((NS, N_PAD), jnp.float32),
        pltpu.SemaphoreType.DMA,
    ],
)(_sc_body)


def _mm_body(x_ref, w_ref, o_ref):
    o_ref[...] = lax.dot_general(
        x_ref[...], w_ref[...], (((1,), (1,)), ((), ())),
        preferred_element_type=jnp.float32)


def _matmul(xp, W):
    return pl.pallas_call(
        _mm_body,
        grid=(N_PAD // 1024,),
        in_specs=[
            pl.BlockSpec((1024, D), lambda i: (i, 0)),
            pl.BlockSpec((D, D), lambda i: (0, 0)),
        ],
        out_specs=pl.BlockSpec((1024, D), lambda i: (i, 0)),
        out_shape=jax.ShapeDtypeStruct((N_PAD, D), jnp.float32),
    )(xp, W)


def _ep_body(p_ref, xw_ref, dis_ref, b_ref, lnw_ref, lnb_ref, o_ref):
    d = dis_ref[...]
    p = p_ref[0] + p_ref[1]
    o = d * p + (d * d) * xw_ref[...] + b_ref[...]
    o = jnp.where(o >= 0, o, 0.01 * o)
    mu = jnp.mean(o, axis=-1, keepdims=True)
    zc = o - mu
    var = jnp.mean(zc * zc, axis=-1, keepdims=True)
    o_ref[...] = zc * lax.rsqrt(var + 1e-5) * lnw_ref[...] + lnb_ref[...]


def _epilogue(part, xw, dis2d, b2d, lnw2d, lnb2d):
    return pl.pallas_call(
        _ep_body,
        grid=(N_PAD // 1024,),
        in_specs=[
            pl.BlockSpec((NC, 1024, D), lambda i: (0, i, 0)),
            pl.BlockSpec((1024, D), lambda i: (i, 0)),
            pl.BlockSpec((1024, 1), lambda i: (i, 0)),
            pl.BlockSpec((1, D), lambda i: (0, 0)),
            pl.BlockSpec((1, D), lambda i: (0, 0)),
            pl.BlockSpec((1, D), lambda i: (0, 0)),
        ],
        out_specs=pl.BlockSpec((1024, D), lambda i: (i, 0)),
        out_shape=jax.ShapeDtypeStruct((N_PAD, D), jnp.float32),
    )(part, xw, dis2d, b2d, lnw2d, lnb2d)


def kernel(x, edge_attr, W, b, ln_w, ln_b, edge_index):
    row = edge_index[0].astype(jnp.int32).reshape(N_CHUNKS, CHUNK)
    col = edge_index[1].astype(jnp.int32).reshape(N_CHUNKS, CHUNK)
    ea2 = edge_attr.astype(jnp.float32).reshape(N_CHUNKS, CHUNK * 4)
    xp = jnp.pad(x.astype(jnp.float32), ((0, N_PAD - N), (0, 0)))
    xw = _matmul(xp, W)
    part, dis = _sc_call(row, col, ea2, xw)
    out = _epilogue(part, xw, dis.reshape(N_PAD, 1), b.reshape(1, D),
                    ln_w.reshape(1, D), ln_b.reshape(1, D))
    return out[:N]


# speedup vs baseline: 30.2127x; 3.1012x over previous
import dataclasses
import functools

import jax
import jax.numpy as jnp
from jax import lax
from jax.experimental import pallas as pl
from jax.experimental.pallas import tpu as pltpu
from jax.experimental.pallas import tpu_sc as plsc

N = 10000
N_PAD = 10240
E = 320000
D = 128
CHUNK = 128
NC = 2
NS = 16
NW = NC * NS
SLOTS = 80
N_CHUNKS = SLOTS * NW
E_PAD = N_CHUNKS * CHUNK
SBLK = 4
DCH = NS * SLOTS
RPT = N_PAD // NS

_sc_mesh = plsc.VectorSubcoreMesh(
    core_axis_name="c", subcore_axis_name="s", num_cores=NC, num_subcores=NS
)


def _lane_bcast(v16, j):
    idx = jnp.full((16, 1), j, dtype=jnp.int32)
    dn = lax.GatherDimensionNumbers(
        offset_dims=(), collapsed_slice_dims=(0,), start_index_map=(0,))
    return lax.gather(v16, idx, dn, (1,),
                      mode=lax.GatherScatterMode.PROMISE_IN_BOUNDS)


def _sc_body(row_hbm, col_hbm, ew_hbm, xw_hbm, part_hbm, dis_hbm, stage_hbm,
             idx_row, idx_col, ew_sl, rows2, degdis_v, tmp_v,
             acc_sh, ssl0, ssl1, gsem0, gsem1):
    c = lax.axis_index("c")
    s = lax.axis_index("s")
    wid = s * NC + c
    iota16 = lax.iota(jnp.int32, 16)
    zero16 = jnp.zeros((16,), jnp.float32)
    ssl = [ssl0, ssl1]
    gsem = [gsem0, gsem1]

    with jax.named_scope("p0_zero"):
        @pl.loop(0, N_PAD // 16)
        def _(i):
            degdis_v[pl.ds(i * 16, 16)] = zero16

        @pl.loop(0, CHUNK)
        def _(i):
            @pl.loop(0, D // 16)
            def _(r):
                rows2[0, i, pl.ds(r * 16, 16)] = zero16

        @pl.loop(0, RPT // CHUNK)
        def _(i):
            pltpu.sync_copy(rows2.at[0],
                            acc_sh.at[pl.ds(s * RPT + i * CHUNK, CHUNK)])

    dpt = N_CHUNKS // NS
    dnb = dpt // SBLK
    dbase = s * dpt

    def deg_load(u, blk):
        pltpu.async_copy(col_hbm.at[pl.ds(dbase + blk * SBLK, SBLK), :],
                         idx_col.at[u], ssl[u])
        pltpu.async_copy(ew_hbm.at[pl.ds(dbase + blk * SBLK, SBLK), :],
                         ew_sl.at[u], ssl[u])

    def deg_wait(u, blk):
        pltpu.make_async_copy(col_hbm.at[pl.ds(dbase + blk * SBLK, SBLK), :],
                              idx_col.at[u], ssl[u]).wait()
        pltpu.make_async_copy(ew_hbm.at[pl.ds(dbase + blk * SBLK, SBLK), :],
                              ew_sl.at[u], ssl[u]).wait()

    def deg_compute(u):
        @pl.loop(0, SBLK)
        def _(r):
            @pl.loop(0, CHUNK // 16)
            def _(g):
                col16 = idx_col[u, r, pl.ds(g * 16, 16)]
                ew16 = jnp.abs(ew_sl[u, r, pl.ds(g * 16, 16)])
                for j in range(16):
                    plsc.addupdate_scatter(degdis_v, [col16], ew16,
                                           mask=iota16 == j)

    with jax.named_scope("p1_deg"):
        deg_load(0, 0)

        @pl.loop(0, dnb // 2)
        def _(i):
            for u in range(2):
                blk = i * 2 + u

                @pl.when(blk + 1 < dnb)
                def _():
                    deg_load(1 - u, blk + 1)

                deg_wait(u, blk)
                deg_compute(u)

    with jax.named_scope("p2_degreduce"):
        pltpu.sync_copy(degdis_v, stage_hbm.at[c, s])
        plsc.subcore_barrier()

        @pl.loop(0, NS)
        def _(i):
            @pl.when(i != s)
            def _():
                pltpu.sync_copy(stage_hbm.at[c, i, pl.ds(s * RPT, RPT)],
                                tmp_v)

                @pl.loop(0, RPT // 16)
                def _(r):
                    o = s * RPT + r * 16
                    degdis_v[pl.ds(o, 16)] = (degdis_v[pl.ds(o, 16)]
                                              + tmp_v[pl.ds(r * 16, 16)])

    with jax.named_scope("p3_newton"):
        @pl.loop(0, RPT // 16)
        def _(r):
            o = s * RPT + r * 16
            x = degdis_v[pl.ds(o, 16)] + 1.0
            ib = plsc.bitcast(x, jnp.int32)
            ib = 0x5F3759DF - lax.shift_right_logical(ib, 1)
            y = plsc.bitcast(ib, jnp.float32)
            xh = 0.5 * x
            y = y * (1.5 - xh * y * y)
            y = y * (1.5 - xh * y * y)
            y = y * (1.5 - xh * y * y)
            y = y * (1.5 - xh * y * y)
            degdis_v[pl.ds(o, 16)] = y

        plsc.subcore_barrier()
        pltpu.sync_copy(degdis_v.at[pl.ds(s * RPT, RPT)],
                        stage_hbm.at[c, 0, pl.ds(s * RPT, RPT)])

        @pl.when(c == 0)
        def _():
            pltpu.sync_copy(degdis_v.at[pl.ds(s * RPT, RPT)],
                            dis_hbm.at[pl.ds(s * RPT, RPT)])

        plsc.subcore_barrier()
        pltpu.sync_copy(stage_hbm.at[c, 0], degdis_v)

    mbase = wid * SLOTS

    def slab_load(u, sb):
        pltpu.async_copy(row_hbm.at[pl.ds(mbase + sb * SBLK, SBLK), :],
                         idx_row.at[u], ssl[u])
        pltpu.async_copy(col_hbm.at[pl.ds(mbase + sb * SBLK, SBLK), :],
                         idx_col.at[u], ssl[u])
        pltpu.async_copy(ew_hbm.at[pl.ds(mbase + sb * SBLK, SBLK), :],
                         ew_sl.at[u], ssl[u])

    def slab_wait(u, sb):
        pltpu.make_async_copy(row_hbm.at[pl.ds(mbase + sb * SBLK, SBLK), :],
                              idx_row.at[u], ssl[u]).wait()
        pltpu.make_async_copy(col_hbm.at[pl.ds(mbase + sb * SBLK, SBLK), :],
                              idx_col.at[u], ssl[u]).wait()
        pltpu.make_async_copy(ew_hbm.at[pl.ds(mbase + sb * SBLK, SBLK), :],
                              ew_sl.at[u], ssl[u]).wait()

    def start_gather(b, u, t):
        pltpu.async_copy(xw_hbm.at[idx_row.at[u, t]], rows2.at[b], gsem[b])

    def wait_gather(b, u, t):
        pltpu.make_async_copy(xw_hbm.at[idx_row.at[u, t]], rows2.at[b],
                              gsem[b]).wait()

    def compute_scatter(b, u, t):
        @pl.loop(0, CHUNK // 16)
        def _(g):
            row16 = idx_row[u, t, pl.ds(g * 16, 16)]
            ew16 = jnp.abs(ew_sl[u, t, pl.ds(g * 16, 16)])
            disr16 = plsc.load_gather(degdis_v, [row16])
            w16 = ew16 * disr16
            for j in range(16):
                wb = _lane_bcast(w16, j)
                e = g * 16 + j
                for r in range(D // 16):
                    rows2[b, e, pl.ds(r * 16, 16)] = (
                        rows2[b, e, pl.ds(r * 16, 16)] * wb)

        pltpu.sync_copy(rows2.at[b], acc_sh.at[idx_col.at[u, t]], add=True)

    with jax.named_scope("p4_msgs"):
        slab_load(0, 0)
        slab_wait(0, 0)
        start_gather(0, 0, 0)

        @pl.loop(0, SLOTS // (2 * SBLK))
        def _(p):
            for tg in range(2 * SBLK):
                u, t, b = tg // SBLK, tg % SBLK, tg % 2
                if tg == 0:
                    slab_load(1, 2 * p + 1)
                if tg == SBLK:
                    @pl.when(p < SLOTS // (2 * SBLK) - 1)
                    def _():
                        slab_load(0, 2 * p + 2)

                wait_gather(b, u, t)

                if tg < SBLK - 1:
                    start_gather(1 - b, 0, t + 1)
                elif tg == SBLK - 1:
                    slab_wait(1, 2 * p + 1)
                    start_gather(1 - b, 1, 0)
                elif tg < 2 * SBLK - 1:
                    start_gather(1 - b, 1, t + 1)
                else:
                    @pl.when(p < SLOTS // (2 * SBLK) - 1)
                    def _():
                        slab_wait(0, 2 * p + 2)
                        start_gather(1 - b, 0, 0)

                compute_scatter(b, u, t)

        plsc.subcore_barrier()

    with jax.named_scope("p5_drain"):
        pltpu.sync_copy(acc_sh.at[pl.ds(s * RPT, RPT)],
                        part_hbm.at[c, pl.ds(s * RPT, RPT)])


_sc_cp = pltpu.CompilerParams()
if "needs_layout_passes" in pltpu.CompilerParams.__dataclass_fields__:
    _sc_cp = dataclasses.replace(_sc_cp, needs_layout_passes=False)

_sc_call = functools.partial(
    pl.kernel,
    compiler_params=_sc_cp,
    out_type=[
        jax.ShapeDtypeStruct((NC, N_PAD, D), jnp.float32),
        jax.ShapeDtypeStruct((N_PAD,), jnp.float32),
        jax.ShapeDtypeStruct((NC, NS, N_PAD), jnp.float32),
    ],
    mesh=_sc_mesh,
    scratch_types=[
        pltpu.VMEM((2, SBLK, CHUNK), jnp.int32),
        pltpu.VMEM((2, SBLK, CHUNK), jnp.int32),
        pltpu.VMEM((2, SBLK, CHUNK), jnp.float32),
        pltpu.VMEM((2, CHUNK, D), jnp.float32),
        pltpu.VMEM((N_PAD,), jnp.float32),
        pltpu.VMEM((RPT,), jnp.float32),
        pltpu.VMEM_SHARED((N_PAD, D), jnp.float32),
        pltpu.SemaphoreType.DMA,
        pltpu.SemaphoreType.DMA,
        pltpu.SemaphoreType.DMA,
        pltpu.SemaphoreType.DMA,
    ],
)(_sc_body)


def _mm_body(x_ref, w_ref, o_ref):
    o_ref[...] = lax.dot_general(
        x_ref[...], w_ref[...], (((1,), (1,)), ((), ())),
        preferred_element_type=jnp.float32)


def _matmul(xp, W):
    return pl.pallas_call(
        _mm_body,
        grid=(N_PAD // 1024,),
        in_specs=[
            pl.BlockSpec((1024, D), lambda i: (i, 0)),
            pl.BlockSpec((D, D), lambda i: (0, 0)),
        ],
        out_specs=pl.BlockSpec((1024, D), lambda i: (i, 0)),
        out_shape=jax.ShapeDtypeStruct((N_PAD, D), jnp.float32),
    )(xp, W)


def _ep_body(p_ref, xw_ref, dis_ref, b_ref, lnw_ref, lnb_ref, o_ref):
    d = dis_ref[...]
    p = p_ref[0] + p_ref[1]
    o = d * p + (d * d) * xw_ref[...] + b_ref[...]
    o = jnp.where(o >= 0, o, 0.01 * o)
    mu = jnp.mean(o, axis=-1, keepdims=True)
    zc = o - mu
    var = jnp.mean(zc * zc, axis=-1, keepdims=True)
    o_ref[...] = zc * lax.rsqrt(var + 1e-5) * lnw_ref[...] + lnb_ref[...]


def _epilogue(part, xw, dis2d, b2d, lnw2d, lnb2d):
    return pl.pallas_call(
        _ep_body,
        grid=(N_PAD // 1024,),
        in_specs=[
            pl.BlockSpec((NC, 1024, D), lambda i: (0, i, 0)),
            pl.BlockSpec((1024, D), lambda i: (i, 0)),
            pl.BlockSpec((1024, 1), lambda i: (i, 0)),
            pl.BlockSpec((1, D), lambda i: (0, 0)),
            pl.BlockSpec((1, D), lambda i: (0, 0)),
            pl.BlockSpec((1, D), lambda i: (0, 0)),
        ],
        out_specs=pl.BlockSpec((1024, D), lambda i: (i, 0)),
        out_shape=jax.ShapeDtypeStruct((N_PAD, D), jnp.float32),
    )(part, xw, dis2d, b2d, lnw2d, lnb2d)


def kernel(x, edge_attr, W, b, ln_w, ln_b, edge_index):
    pad_e = E_PAD - E
    spread = (jnp.arange(pad_e, dtype=jnp.int32) * 37) % N
    row = jnp.concatenate([edge_index[0].astype(jnp.int32), spread])
    col = jnp.concatenate([edge_index[1].astype(jnp.int32), spread])
    row = row.reshape(N_CHUNKS, CHUNK)
    col = col.reshape(N_CHUNKS, CHUNK)
    ew = jnp.pad(edge_attr[:, 0].astype(jnp.float32), (0, pad_e))
    ew = ew.reshape(N_CHUNKS, CHUNK)
    xp = jnp.pad(x.astype(jnp.float32), ((0, N_PAD - N), (0, 0)))
    xw = _matmul(xp, W)
    part, dis, _ = _sc_call(row, col, ew, xw)
    out = _epilogue(part, xw, dis.reshape(N_PAD, 1), b.reshape(1, D),
                    ln_w.reshape(1, D), ln_b.reshape(1, D))
    return out[:N]
